# Initial kernel scaffold; baseline (speedup 1.0000x reference)
import jax
import jax.numpy as jnp
from jax.experimental import pallas as pl


def kernel(input_tensor, data, a, b):
    raise NotImplementedError("write your pallas kernel here")


# speedup vs baseline: 99.6672x; 99.6672x over previous
import dataclasses
import functools

import jax
import jax.numpy as jnp
from jax import lax
from jax.experimental import pallas as pl
from jax.experimental.pallas import tpu as pltpu
from jax.experimental.pallas import tpu_sc as plsc

_SIZE = 1048576
_P = 2147483647
_CHUNK = 8
_NHASH = 8
_BATCH = 4096
_FEAT = 26
_M = _BATCH * _FEAT
_NCHUNKS = _M * _NHASH
_NW = 32
_CPW = _NCHUNKS // _NW
_XPW = _M // _NW
_NB = 4
_K = _CPW // _NB
_KR = _K // 128

_CP = pltpu.CompilerParams()
if "needs_layout_passes" in pltpu.CompilerParams.__dataclass_fields__:
    _CP = dataclasses.replace(_CP, needs_layout_passes=False)
if "use_tc_tiling_on_sc" in pltpu.CompilerParams.__dataclass_fields__:
    _CP = dataclasses.replace(_CP, use_tc_tiling_on_sc=False)


@functools.partial(
    pl.kernel,
    out_type=jax.ShapeDtypeStruct((_NCHUNKS // 128, 128, _CHUNK), jnp.float32),
    mesh=plsc.VectorSubcoreMesh(core_axis_name="c", subcore_axis_name="s"),
    scratch_types=[
        pltpu.VMEM((_XPW,), jnp.int32),
        pltpu.VMEM((16,), jnp.int32),
        pltpu.VMEM((_KR, 128), jnp.int32),
        pltpu.VMEM((_KR, 128, _CHUNK), jnp.float32),
        pltpu.SemaphoreType.DMA,
    ],
    compiler_params=_CP,
)
def _robe_sc(x_hbm, w_hbm, ab_hbm, out_hbm, x_v, ab_v, idx_v, rows_v, sem):
    wid = lax.axis_index("s") * 2 + lax.axis_index("c")
    pltpu.sync_copy(x_hbm.at[pl.ds(wid * _XPW, _XPW)], x_v)
    pltpu.sync_copy(ab_hbm, ab_v)
    lane = lax.iota(jnp.int32, 16)
    k8 = lane & 7
    av = plsc.load_gather(ab_v, [k8])
    bv = plsc.load_gather(ab_v, [k8 + 8])

    @pl.loop(0, _NB)
    def _blk(blk):
        cbase = blk * _K

        @pl.loop(0, _KR)
        def _hash_row(j):
            @pl.loop(0, 8)
            def _grp(q):
                c = (cbase + j * 128 + q * 16) + lane
                xv = plsc.load_gather(x_v, [c >> 3])
                v = xv * av + bv
                v = jnp.where(v < 0, v + _P, v)
                v = jnp.where(v < 0, v + _P, v)
                v = jnp.where(v >= _P, v - _P, v)
                idx_v[j, pl.ds(q * 16, 16)] = v & (_SIZE - 1)

        @pl.loop(0, _KR)
        def _fire(j):
            pltpu.async_copy(w_hbm.at[idx_v.at[j]], rows_v.at[j], sem)

        @pl.loop(0, _KR)
        def _drain(j):
            pltpu.make_async_copy(w_hbm.at[idx_v.at[j]], rows_v.at[j], sem).wait()

        pltpu.sync_copy(
            rows_v, out_hbm.at[pl.ds(wid * (_CPW // 128) + blk * _KR, _KR)]
        )


def kernel(input_tensor, data, a, b):
    x_flat = input_tensor.reshape(-1)
    data_ext = jnp.concatenate([data, jnp.broadcast_to(data[-1], (_CHUNK,))])
    w = jnp.stack([data_ext[s:s + _SIZE] for s in range(_CHUNK)], axis=1)
    ab = jnp.concatenate([a, b])
    out = _robe_sc(x_flat, w, ab)
    return out.reshape(_BATCH, _FEAT, _NHASH * _CHUNK)
